# baseline (speedup 1.0000x reference)
import functools

import jax
import jax.numpy as jnp
from jax import lax
from jax.experimental import pallas as pl
from jax.experimental.pallas import tpu as pltpu
from jax.experimental.pallas import tpu_sc as plsc

HIDDEN = 256
FF = 640
E = 3
TG = 512
T = 512
NW = 32
CH = 128


def _gating_kernel(x_ref, gw_ref, gb_ref,
                   eid_ref, rank_ref, w_ref, counts_ref, scores_ref):
    i = pl.program_id(0)

    @pl.when(i == 0)
    def _init():
        for e in range(E):
            counts_ref[e] = 0.0
            scores_ref[e] = 0.0

    x = x_ref[...]
    xb = x.astype(jnp.bfloat16)
    logits = jnp.dot(xb, gw_ref[...], preferred_element_type=jnp.float32)
    l0 = logits[:, 0] + gb_ref[0]
    l1 = logits[:, 1] + gb_ref[1]
    l2 = logits[:, 2] + gb_ref[2]
    m = jnp.maximum(jnp.maximum(l0, l1), l2)
    u0, u1, u2 = jnp.exp(l0 - m), jnp.exp(l1 - m), jnp.exp(l2 - m)
    s = u0 + u1 + u2
    p0, p1, p2 = u0 / s, u1 / s, u2 / s
    w = jnp.maximum(jnp.maximum(p0, p1), p2)
    eid = jnp.where(p0 >= p1,
                    jnp.where(p0 >= p2, 0, 2),
                    jnp.where(p1 >= p2, 1, 2)).astype(jnp.int32)

    eidx = lax.broadcasted_iota(jnp.int32, (TG, E), 1)
    onehot = (eid[:, None] == eidx).astype(jnp.float32)
    row = lax.broadcasted_iota(jnp.int32, (TG, TG), 0)
    col = lax.broadcasted_iota(jnp.int32, (TG, TG), 1)
    ltri = (col < row).astype(jnp.bfloat16)
    prefix = jnp.dot(ltri, onehot.astype(jnp.bfloat16),
                     preferred_element_type=jnp.float32)

    carry = (counts_ref[0] * onehot[:, 0] + counts_ref[1] * onehot[:, 1]
             + counts_ref[2] * onehot[:, 2])
    rank = jnp.sum(prefix * onehot, axis=1) + carry

    eid_ref[0, 0, :] = eid
    rank_ref[0, 0, :] = rank.astype(jnp.int32)
    w_ref[0, 0, :] = w

    for e in range(E):
        counts_ref[e] += jnp.sum(onehot[:, e])
        scores_ref[e] += jnp.sum(jnp.where(eid == e, w, 0.0))


def _gating(x_flat, gate_Wt, gate_b):
    N = x_flat.shape[0]
    NB = N // TG
    return pl.pallas_call(
        _gating_kernel,
        grid=(NB,),
        in_specs=[
            pl.BlockSpec((TG, HIDDEN), lambda i: (i, 0)),
            pl.BlockSpec((HIDDEN, E), lambda i: (0, 0)),
            pl.BlockSpec(memory_space=pltpu.SMEM),
        ],
        out_specs=[
            pl.BlockSpec((1, 1, TG), lambda i: (i, 0, 0)),
            pl.BlockSpec((1, 1, TG), lambda i: (i, 0, 0)),
            pl.BlockSpec((1, 1, TG), lambda i: (i, 0, 0)),
            pl.BlockSpec(memory_space=pltpu.SMEM),
            pl.BlockSpec(memory_space=pltpu.SMEM),
        ],
        out_shape=[
            jax.ShapeDtypeStruct((NB, 1, TG), jnp.int32),
            jax.ShapeDtypeStruct((NB, 1, TG), jnp.int32),
            jax.ShapeDtypeStruct((NB, 1, TG), jnp.float32),
            jax.ShapeDtypeStruct((E,), jnp.float32),
            jax.ShapeDtypeStruct((E,), jnp.float32),
        ],
    )(x_flat, gate_Wt, gate_b)


def _sc_pos_body(eid_v, rank_v, ps1_v, ps2_v, idx_v):
    @pl.loop(0, CH, step=16)
    def _(g):
        ev = eid_v[pl.ds(g, 16)]
        rv = rank_v[pl.ds(g, 16)]
        z = jnp.zeros((16,), jnp.int32)
        pos = (rv + jnp.where(ev == 1, ps1_v[...], z)
               + jnp.where(ev == 2, ps2_v[...], z))
        idx_v[pl.ds(g, 16)] = pos


def _dispatch(x_flat, eid, rank, ps1, ps2, P):
    N, d = x_flat.shape
    BPW = N // NW
    mesh = plsc.VectorSubcoreMesh(core_axis_name="c", subcore_axis_name="s")

    @functools.partial(
        pl.kernel, mesh=mesh,
        out_type=jax.ShapeDtypeStruct((P, d), jnp.float32),
        scratch_types=[
            pltpu.VMEM((CH,), jnp.int32),
            pltpu.VMEM((CH,), jnp.int32),
            pltpu.VMEM((CH,), jnp.int32),
            pltpu.VMEM((16,), jnp.int32),
            pltpu.VMEM((16,), jnp.int32),
            pltpu.VMEM((CH, d), jnp.float32),
        ],
    )
    def k(x_hbm, eid_hbm, rank_hbm, ps1_hbm, ps2_hbm, xs_hbm,
          eid_v, rank_v, idx_v, ps1_v, ps2_v, rows_v):
        wid = lax.axis_index("s") * 2 + lax.axis_index("c")
        base = wid * BPW
        pltpu.sync_copy(ps1_hbm, ps1_v)
        pltpu.sync_copy(ps2_hbm, ps2_v)

        @pl.loop(0, BPW, step=CH)
        def _(c):
            row0 = base + c
            pltpu.sync_copy(eid_hbm.at[pl.ds(row0, CH)], eid_v)
            pltpu.sync_copy(rank_hbm.at[pl.ds(row0, CH)], rank_v)
            _sc_pos_body(eid_v, rank_v, ps1_v, ps2_v, idx_v)
            pltpu.sync_copy(x_hbm.at[pl.ds(row0, CH)], rows_v)
            pltpu.sync_copy(rows_v, xs_hbm.at[idx_v])

    return k(x_flat, eid, rank, ps1, ps2)


def _combine(ys, eid, rank, ps1, ps2, N):
    P, d = ys.shape
    BPW = N // NW
    mesh = plsc.VectorSubcoreMesh(core_axis_name="c", subcore_axis_name="s")

    @functools.partial(
        pl.kernel, mesh=mesh,
        out_type=jax.ShapeDtypeStruct((N, d), jnp.float32),
        scratch_types=[
            pltpu.VMEM((CH,), jnp.int32),
            pltpu.VMEM((CH,), jnp.int32),
            pltpu.VMEM((CH,), jnp.int32),
            pltpu.VMEM((16,), jnp.int32),
            pltpu.VMEM((16,), jnp.int32),
            pltpu.VMEM((CH, d), jnp.float32),
        ],
    )
    def k(ys_hbm, eid_hbm, rank_hbm, ps1_hbm, ps2_hbm, g_hbm,
          eid_v, rank_v, idx_v, ps1_v, ps2_v, rows_v):
        wid = lax.axis_index("s") * 2 + lax.axis_index("c")
        base = wid * BPW
        pltpu.sync_copy(ps1_hbm, ps1_v)
        pltpu.sync_copy(ps2_hbm, ps2_v)

        @pl.loop(0, BPW, step=CH)
        def _(c):
            row0 = base + c
            pltpu.sync_copy(eid_hbm.at[pl.ds(row0, CH)], eid_v)
            pltpu.sync_copy(rank_hbm.at[pl.ds(row0, CH)], rank_v)
            _sc_pos_body(eid_v, rank_v, ps1_v, ps2_v, idx_v)
            pltpu.sync_copy(ys_hbm.at[idx_v], rows_v)
            pltpu.sync_copy(rows_v, g_hbm.at[pl.ds(row0, CH)])

    return k(ys, eid, rank, ps1, ps2)


def _expert_kernel(eids_ref, xs_ref, fcw_ref, fcb_ref, ow_ref, ob_ref, ys_ref):
    xb = xs_ref[...].astype(jnp.bfloat16)
    h = jnp.dot(xb, fcw_ref[0], preferred_element_type=jnp.float32)
    h = h + fcb_ref[0]
    x1 = h[:, :FF]
    x2 = h[:, FF:]
    g = x1 * (0.5 * x2 * (1.0 + lax.erf(x2 * 0.7071067811865476)))
    y = jnp.dot(g.astype(jnp.bfloat16), ow_ref[0],
                preferred_element_type=jnp.float32)
    ys_ref[...] = y + ob_ref[0]


def _experts(xs, eids, fc_Wt, fc_b, out_Wt, out_b):
    P, d = xs.shape
    NT = P // T
    grid_spec = pltpu.PrefetchScalarGridSpec(
        num_scalar_prefetch=1,
        grid=(NT,),
        in_specs=[
            pl.BlockSpec((T, d), lambda i, eids: (i, 0)),
            pl.BlockSpec((1, d, 2 * FF), lambda i, eids: (eids[i], 0, 0)),
            pl.BlockSpec((1, 1, 2 * FF), lambda i, eids: (eids[i], 0, 0)),
            pl.BlockSpec((1, FF, d), lambda i, eids: (eids[i], 0, 0)),
            pl.BlockSpec((1, 1, d), lambda i, eids: (eids[i], 0, 0)),
        ],
        out_specs=pl.BlockSpec((T, d), lambda i, eids: (i, 0)),
    )
    return pl.pallas_call(
        _expert_kernel,
        grid_spec=grid_spec,
        out_shape=jax.ShapeDtypeStruct((P, d), jnp.float32),
    )(eids, xs, fc_Wt, fc_b.reshape(E, 1, 2 * FF), out_Wt,
      out_b.reshape(E, 1, HIDDEN))


def _scale_kernel(g_ref, w_ref, out_ref):
    wt = jnp.transpose(w_ref[0], (1, 0))
    out_ref[...] = g_ref[...] * wt


def _scale(gathered, w3):
    N, d = gathered.shape
    NB = N // TG
    return pl.pallas_call(
        _scale_kernel,
        grid=(NB,),
        in_specs=[
            pl.BlockSpec((TG, d), lambda i: (i, 0)),
            pl.BlockSpec((1, 1, TG), lambda i: (i, 0, 0)),
        ],
        out_specs=pl.BlockSpec((TG, d), lambda i: (i, 0)),
        out_shape=jax.ShapeDtypeStruct((N, d), jnp.float32),
    )(gathered, w3)


@jax.jit
def kernel(x, gate_W, gate_b, fc_W, fc_b, out_W, out_b):
    B, S, d = x.shape
    N = B * S
    P = N + E * T
    x_flat = x.reshape(N, d)
    gate_Wt = gate_W.T.astype(jnp.bfloat16)
    fc_Wt = fc_W.transpose(0, 2, 1).astype(jnp.bfloat16)
    out_Wt = out_W.transpose(0, 2, 1).astype(jnp.bfloat16)

    eid3, rank3, w3, counts, scores = _gating(x_flat, gate_Wt, gate_b)
    eid = eid3.reshape(N)
    rank = rank3.reshape(N)

    counts_i = counts.astype(jnp.int32)
    cpad = ((counts_i + T - 1) // T) * T
    ps1 = cpad[0]
    ps2 = cpad[0] + cpad[1]
    ps1v = jnp.full((16,), 1, jnp.int32) * ps1
    ps2v = jnp.full((16,), 1, jnp.int32) * ps2
    NT = P // T
    t_idx = jnp.arange(NT, dtype=jnp.int32)
    eids = ((t_idx >= ps1 // T).astype(jnp.int32)
            + (t_idx >= ps2 // T).astype(jnp.int32))

    xs = _dispatch(x_flat, eid, rank, ps1v, ps2v, P)
    ys = _experts(xs, eids, fc_Wt, fc_b, out_Wt, out_b)
    gathered = _combine(ys, eid, rank, ps1v, ps2v, N)
    out = _scale(gathered, w3)

    usage = scores / (counts + 1e-08)
    loss = jnp.sum((usage - 1.0 / E) ** 2)
    return out.reshape(B, S, d), loss


# speedup vs baseline: 1.4760x; 1.4760x over previous
import functools

import jax
import jax.numpy as jnp
from jax import lax
from jax.experimental import pallas as pl
from jax.experimental.pallas import tpu as pltpu
from jax.experimental.pallas import tpu_sc as plsc

HIDDEN = 256
FF = 640
E = 3
TG = 512
T = 1024
NW = 32
CH = 128


def _gating_kernel(x_ref, gw_ref, gb_ref,
                   eid_ref, rank_ref, w_ref, counts_ref, scores_ref,
                   utri_ref):
    i = pl.program_id(0)

    @pl.when(i == 0)
    def _init():
        for e in range(E):
            counts_ref[e] = 0.0
            scores_ref[e] = 0.0
        r = lax.broadcasted_iota(jnp.int32, (TG, TG), 0)
        c = lax.broadcasted_iota(jnp.int32, (TG, TG), 1)
        utri_ref[...] = (r < c).astype(jnp.bfloat16)

    x = x_ref[...]
    xb = x.astype(jnp.bfloat16)
    logits = jnp.dot(xb, gw_ref[...], preferred_element_type=jnp.float32)
    lt = jnp.transpose(logits, (1, 0))
    l0 = lt[0:1, :] + gb_ref[0]
    l1 = lt[1:2, :] + gb_ref[1]
    l2 = lt[2:3, :] + gb_ref[2]
    m = jnp.maximum(jnp.maximum(l0, l1), l2)
    s = jnp.exp(l0 - m) + jnp.exp(l1 - m) + jnp.exp(l2 - m)
    w = 1.0 / s
    eid = jnp.where(l0 >= l1,
                    jnp.where(l0 >= l2, 0, 2),
                    jnp.where(l1 >= l2, 1, 2)).astype(jnp.int32)

    srow = lax.broadcasted_iota(jnp.int32, (E, TG), 0)
    ohT = (jnp.broadcast_to(eid, (E, TG)) == srow).astype(jnp.float32)
    prefix = jnp.dot(ohT.astype(jnp.bfloat16), utri_ref[...],
                     preferred_element_type=jnp.float32)

    rank = jnp.zeros((1, TG), jnp.float32)
    for e in range(E):
        rank = rank + ohT[e:e + 1, :] * (prefix[e:e + 1, :] + counts_ref[e])

    eid_ref[0] = eid
    rank_ref[0] = rank.astype(jnp.int32)
    w_ref[0] = w

    for e in range(E):
        counts_ref[e] += jnp.sum(ohT[e:e + 1, :])
        scores_ref[e] += jnp.sum(ohT[e:e + 1, :] * w)


def _gating(x_flat, gate_Wt, gate_b):
    N = x_flat.shape[0]
    NB = N // TG
    return pl.pallas_call(
        _gating_kernel,
        grid=(NB,),
        in_specs=[
            pl.BlockSpec((TG, HIDDEN), lambda i: (i, 0)),
            pl.BlockSpec((HIDDEN, E), lambda i: (0, 0)),
            pl.BlockSpec(memory_space=pltpu.SMEM),
        ],
        out_specs=[
            pl.BlockSpec((1, 1, TG), lambda i: (i, 0, 0)),
            pl.BlockSpec((1, 1, TG), lambda i: (i, 0, 0)),
            pl.BlockSpec((1, 1, TG), lambda i: (i, 0, 0)),
            pl.BlockSpec(memory_space=pltpu.SMEM),
            pl.BlockSpec(memory_space=pltpu.SMEM),
        ],
        out_shape=[
            jax.ShapeDtypeStruct((NB, 1, TG), jnp.int32),
            jax.ShapeDtypeStruct((NB, 1, TG), jnp.int32),
            jax.ShapeDtypeStruct((NB, 1, TG), jnp.float32),
            jax.ShapeDtypeStruct((E,), jnp.float32),
            jax.ShapeDtypeStruct((E,), jnp.float32),
        ],
        scratch_shapes=[pltpu.VMEM((TG, TG), jnp.bfloat16)],
    )(x_flat, gate_Wt, gate_b)


def _sc_pos_body(eid_v, rank_v, ps1_v, ps2_v, idx_v):
    @pl.loop(0, CH, step=16)
    def _(g):
        ev = eid_v[pl.ds(g, 16)]
        rv = rank_v[pl.ds(g, 16)]
        z = jnp.zeros((16,), jnp.int32)
        pos = (rv + jnp.where(ev == 1, ps1_v[...], z)
               + jnp.where(ev == 2, ps2_v[...], z))
        idx_v[pl.ds(g, 16)] = pos


def _dispatch(x_flat, eid, rank, ps1, ps2, P):
    N, d = x_flat.shape
    BPW = N // NW
    mesh = plsc.VectorSubcoreMesh(core_axis_name="c", subcore_axis_name="s")

    @functools.partial(
        pl.kernel, mesh=mesh,
        out_type=jax.ShapeDtypeStruct((P, d), jnp.float32),
        scratch_types=[
            pltpu.VMEM((CH,), jnp.int32),
            pltpu.VMEM((CH,), jnp.int32),
            pltpu.VMEM((CH,), jnp.int32),
            pltpu.VMEM((16,), jnp.int32),
            pltpu.VMEM((16,), jnp.int32),
            pltpu.VMEM((CH, d), jnp.float32),
        ],
    )
    def k(x_hbm, eid_hbm, rank_hbm, ps1_hbm, ps2_hbm, xs_hbm,
          eid_v, rank_v, idx_v, ps1_v, ps2_v, rows_v):
        wid = lax.axis_index("s") * 2 + lax.axis_index("c")
        base = wid * BPW
        pltpu.sync_copy(ps1_hbm, ps1_v)
        pltpu.sync_copy(ps2_hbm, ps2_v)

        @pl.loop(0, BPW, step=CH)
        def _(c):
            row0 = base + c
            pltpu.sync_copy(eid_hbm.at[pl.ds(row0, CH)], eid_v)
            pltpu.sync_copy(rank_hbm.at[pl.ds(row0, CH)], rank_v)
            _sc_pos_body(eid_v, rank_v, ps1_v, ps2_v, idx_v)
            pltpu.sync_copy(x_hbm.at[pl.ds(row0, CH)], rows_v)
            pltpu.sync_copy(rows_v, xs_hbm.at[idx_v])

    return k(x_flat, eid, rank, ps1, ps2)


def _combine(ys, eid, rank, ps1, ps2, N):
    P, d = ys.shape
    BPW = N // NW
    mesh = plsc.VectorSubcoreMesh(core_axis_name="c", subcore_axis_name="s")

    @functools.partial(
        pl.kernel, mesh=mesh,
        out_type=jax.ShapeDtypeStruct((N, d), jnp.float32),
        scratch_types=[
            pltpu.VMEM((CH,), jnp.int32),
            pltpu.VMEM((CH,), jnp.int32),
            pltpu.VMEM((CH,), jnp.int32),
            pltpu.VMEM((16,), jnp.int32),
            pltpu.VMEM((16,), jnp.int32),
            pltpu.VMEM((CH, d), jnp.float32),
        ],
    )
    def k(ys_hbm, eid_hbm, rank_hbm, ps1_hbm, ps2_hbm, g_hbm,
          eid_v, rank_v, idx_v, ps1_v, ps2_v, rows_v):
        wid = lax.axis_index("s") * 2 + lax.axis_index("c")
        base = wid * BPW
        pltpu.sync_copy(ps1_hbm, ps1_v)
        pltpu.sync_copy(ps2_hbm, ps2_v)

        @pl.loop(0, BPW, step=CH)
        def _(c):
            row0 = base + c
            pltpu.sync_copy(eid_hbm.at[pl.ds(row0, CH)], eid_v)
            pltpu.sync_copy(rank_hbm.at[pl.ds(row0, CH)], rank_v)
            _sc_pos_body(eid_v, rank_v, ps1_v, ps2_v, idx_v)
            pltpu.sync_copy(ys_hbm.at[idx_v], rows_v)
            pltpu.sync_copy(rows_v, g_hbm.at[pl.ds(row0, CH)])

    return k(ys, eid, rank, ps1, ps2)


def _expert_kernel(eids_ref, xs_ref, fcw_ref, fcb_ref, ow_ref, ob_ref, ys_ref):
    xb = xs_ref[...].astype(jnp.bfloat16)
    h = jnp.dot(xb, fcw_ref[0], preferred_element_type=jnp.float32)
    h = h + fcb_ref[0]
    x1 = h[:, :FF]
    x2 = h[:, FF:]
    g = x1 * (0.5 * x2 * (1.0 + lax.erf(x2 * 0.7071067811865476)))
    y = jnp.dot(g.astype(jnp.bfloat16), ow_ref[0],
                preferred_element_type=jnp.float32)
    ys_ref[...] = y + ob_ref[0]


def _experts(xs, eids, fc_Wt, fc_b, out_Wt, out_b):
    P, d = xs.shape
    NT = P // T
    grid_spec = pltpu.PrefetchScalarGridSpec(
        num_scalar_prefetch=1,
        grid=(NT,),
        in_specs=[
            pl.BlockSpec((T, d), lambda i, eids: (i, 0)),
            pl.BlockSpec((1, d, 2 * FF), lambda i, eids: (eids[i], 0, 0)),
            pl.BlockSpec((1, 1, 2 * FF), lambda i, eids: (eids[i], 0, 0)),
            pl.BlockSpec((1, FF, d), lambda i, eids: (eids[i], 0, 0)),
            pl.BlockSpec((1, 1, d), lambda i, eids: (eids[i], 0, 0)),
        ],
        out_specs=pl.BlockSpec((T, d), lambda i, eids: (i, 0)),
    )
    return pl.pallas_call(
        _expert_kernel,
        grid_spec=grid_spec,
        out_shape=jax.ShapeDtypeStruct((P, d), jnp.float32),
    )(eids, xs, fc_Wt, fc_b.reshape(E, 1, 2 * FF), out_Wt,
      out_b.reshape(E, 1, HIDDEN))


def _scale_kernel(g_ref, w_ref, out_ref):
    wt = jnp.transpose(w_ref[0], (1, 0))
    out_ref[...] = g_ref[...] * wt


def _scale(gathered, w3):
    N, d = gathered.shape
    NB = N // TG
    return pl.pallas_call(
        _scale_kernel,
        grid=(NB,),
        in_specs=[
            pl.BlockSpec((TG, d), lambda i: (i, 0)),
            pl.BlockSpec((1, 1, TG), lambda i: (i, 0, 0)),
        ],
        out_specs=pl.BlockSpec((TG, d), lambda i: (i, 0)),
        out_shape=jax.ShapeDtypeStruct((N, d), jnp.float32),
    )(gathered, w3)


@jax.jit
def kernel(x, gate_W, gate_b, fc_W, fc_b, out_W, out_b):
    B, S, d = x.shape
    N = B * S
    P = N + E * T
    x_flat = x.reshape(N, d)
    gate_Wt = gate_W.T.astype(jnp.bfloat16)
    fc_Wt = fc_W.transpose(0, 2, 1).astype(jnp.bfloat16)
    out_Wt = out_W.transpose(0, 2, 1).astype(jnp.bfloat16)

    eid3, rank3, w3, counts, scores = _gating(x_flat, gate_Wt, gate_b)
    eid = eid3.reshape(N)
    rank = rank3.reshape(N)

    counts_i = counts.astype(jnp.int32)
    cpad = ((counts_i + T - 1) // T) * T
    ps1 = cpad[0]
    ps2 = cpad[0] + cpad[1]
    ps1v = jnp.full((16,), 1, jnp.int32) * ps1
    ps2v = jnp.full((16,), 1, jnp.int32) * ps2
    NT = P // T
    t_idx = jnp.arange(NT, dtype=jnp.int32)
    eids = ((t_idx >= ps1 // T).astype(jnp.int32)
            + (t_idx >= ps2 // T).astype(jnp.int32))

    xs = _dispatch(x_flat, eid, rank, ps1v, ps2v, P)
    ys = _experts(xs, eids, fc_Wt, fc_b, out_Wt, out_b)
    gathered = _combine(ys, eid, rank, ps1v, ps2v, N)
    out = _scale(gathered, w3)

    usage = scores / (counts + 1e-08)
    loss = jnp.sum((usage - 1.0 / E) ** 2)
    return out.reshape(B, S, d), loss


# speedup vs baseline: 1.6157x; 1.0947x over previous
import functools

import jax
import jax.numpy as jnp
from jax import lax
from jax.experimental import pallas as pl
from jax.experimental.pallas import tpu as pltpu
from jax.experimental.pallas import tpu_sc as plsc

HIDDEN = 256
FF = 640
E = 3
TG = 512
T = 1024
NW = 32
CH = 128


def _gating_kernel(x_ref, gw_ref, gb_ref,
                   eid_ref, rank_ref, w16_ref, counts_ref, scores_ref,
                   utri_ref):
    i = pl.program_id(0)

    @pl.when(i == 0)
    def _init():
        for e in range(E):
            counts_ref[e] = 0.0
            scores_ref[e] = 0.0
        r = lax.broadcasted_iota(jnp.int32, (TG, TG), 0)
        c = lax.broadcasted_iota(jnp.int32, (TG, TG), 1)
        utri_ref[...] = (r < c).astype(jnp.bfloat16)

    x = x_ref[...]
    xb = x.astype(jnp.bfloat16)
    logits = jnp.dot(xb, gw_ref[...], preferred_element_type=jnp.float32)
    lt = jnp.transpose(logits, (1, 0))
    l0 = lt[0:1, :] + gb_ref[0]
    l1 = lt[1:2, :] + gb_ref[1]
    l2 = lt[2:3, :] + gb_ref[2]
    m = jnp.maximum(jnp.maximum(l0, l1), l2)
    s = jnp.exp(l0 - m) + jnp.exp(l1 - m) + jnp.exp(l2 - m)
    w = 1.0 / s
    eid = jnp.where(l0 >= l1,
                    jnp.where(l0 >= l2, 0, 2),
                    jnp.where(l1 >= l2, 1, 2)).astype(jnp.int32)

    srow = lax.broadcasted_iota(jnp.int32, (E, TG), 0)
    ohT = (jnp.broadcast_to(eid, (E, TG)) == srow).astype(jnp.float32)
    prefix = jnp.dot(ohT.astype(jnp.bfloat16), utri_ref[...],
                     preferred_element_type=jnp.float32)

    rank = jnp.zeros((1, TG), jnp.float32)
    for e in range(E):
        rank = rank + ohT[e:e + 1, :] * (prefix[e:e + 1, :] + counts_ref[e])

    eid_ref[0] = eid
    rank_ref[0] = rank.astype(jnp.int32)
    w16_ref[0] = w

    for e in range(E):
        counts_ref[e] += jnp.sum(ohT[e:e + 1, :])
        scores_ref[e] += jnp.sum(ohT[e:e + 1, :] * w)


def _gating(x_flat, gate_Wt, gate_b):
    N = x_flat.shape[0]
    NB = N // TG
    return pl.pallas_call(
        _gating_kernel,
        grid=(NB,),
        in_specs=[
            pl.BlockSpec((TG, HIDDEN), lambda i: (i, 0)),
            pl.BlockSpec((HIDDEN, E), lambda i: (0, 0)),
            pl.BlockSpec(memory_space=pltpu.SMEM),
        ],
        out_specs=[
            pl.BlockSpec((1, 1, TG), lambda i: (i, 0, 0)),
            pl.BlockSpec((1, 1, TG), lambda i: (i, 0, 0)),
            pl.BlockSpec((1, 1, TG), lambda i: (i, 0, 0)),
            pl.BlockSpec(memory_space=pltpu.SMEM),
            pl.BlockSpec(memory_space=pltpu.SMEM),
        ],
        out_shape=[
            jax.ShapeDtypeStruct((NB, 1, TG), jnp.int32),
            jax.ShapeDtypeStruct((NB, 1, TG), jnp.int32),
            jax.ShapeDtypeStruct((NB, 1, TG), jnp.float32),
            jax.ShapeDtypeStruct((E,), jnp.float32),
            jax.ShapeDtypeStruct((E,), jnp.float32),
        ],
        scratch_shapes=[pltpu.VMEM((TG, TG), jnp.bfloat16)],
    )(x_flat, gate_Wt, gate_b)


def _sc_idx_table(eid_v, rank_v, ps1_v, ps2_v, idx_buf, bpw):
    @pl.loop(0, bpw // CH)
    def _(c):
        @pl.loop(0, CH, step=16)
        def _(g):
            ev = eid_v[pl.ds(c * CH + g, 16)]
            rv = rank_v[pl.ds(c * CH + g, 16)]
            z = jnp.zeros((16,), jnp.int32)
            pos = (rv + jnp.where(ev == 1, ps1_v[...], z)
                   + jnp.where(ev == 2, ps2_v[...], z))
            idx_buf.at[c][pl.ds(g, 16)] = pos


def _dispatch(x_flat, eid, rank, ps1, ps2, P):
    N, d = x_flat.shape
    BPW = N // NW
    mesh = plsc.VectorSubcoreMesh(core_axis_name="c", subcore_axis_name="s")

    @functools.partial(
        pl.kernel, mesh=mesh,
        out_type=jax.ShapeDtypeStruct((P, d), jnp.float32),
        scratch_types=[
            pltpu.VMEM((N // NW,), jnp.int32),
            pltpu.VMEM((N // NW,), jnp.int32),
            pltpu.VMEM((N // NW // CH, CH), jnp.int32),
            pltpu.VMEM((16,), jnp.int32),
            pltpu.VMEM((16,), jnp.int32),
            pltpu.VMEM((CH, HIDDEN), jnp.float32),
            pltpu.VMEM((CH, HIDDEN), jnp.float32),
            pltpu.SemaphoreType.DMA,
            pltpu.SemaphoreType.DMA,
            pltpu.SemaphoreType.DMA,
            pltpu.SemaphoreType.DMA,
        ],
    )
    def k(x_hbm, eid_hbm, rank_hbm, ps1_hbm, ps2_hbm, xs_hbm,
          eid_v, rank_v, idx_buf, ps1_v, ps2_v,
          xb0, xb1, semr0, semr1, sems0, sems1):
        wid = lax.axis_index("s") * 2 + lax.axis_index("c")
        base = wid * BPW
        nch = BPW // CH
        xbuf = (xb0, xb1)
        semr = (semr0, semr1)
        sems = (sems0, sems1)
        pltpu.sync_copy(ps1_hbm, ps1_v)
        pltpu.sync_copy(ps2_hbm, ps2_v)
        pltpu.sync_copy(eid_hbm.at[pl.ds(base, BPW)], eid_v)
        pltpu.sync_copy(rank_hbm.at[pl.ds(base, BPW)], rank_v)
        _sc_idx_table(eid_v, rank_v, ps1_v, ps2_v, idx_buf, BPW)

        def rd(b, c):
            row0 = base + c * CH
            return pltpu.make_async_copy(x_hbm.at[pl.ds(row0, CH)],
                                         xbuf[b], semr[b])

        def sc(b, c):
            return pltpu.make_async_copy(xbuf[b], xs_hbm.at[idx_buf.at[c]],
                                         sems[b])

        rd(0, 0).start()
        rd(1, 1).start()

        @pl.loop(0, nch, step=2)
        def _(c0):
            for b in range(2):
                c = c0 + b
                rd(b, c).wait()
                sc(b, c).start()

                @pl.when(c + 2 < nch)
                def _():
                    sc(b, c).wait()
                    rd(b, c + 2).start()

        for b in range(2):
            sc(b, nch - 2 + b).wait()

    return k(x_flat, eid, rank, ps1, ps2)


def _combine(ys, eid, rank, ps1, ps2, N):
    P, d = ys.shape
    BPW = N // NW
    mesh = plsc.VectorSubcoreMesh(core_axis_name="c", subcore_axis_name="s")

    @functools.partial(
        pl.kernel, mesh=mesh,
        out_type=jax.ShapeDtypeStruct((N, d), jnp.float32),
        scratch_types=[
            pltpu.VMEM((N // NW,), jnp.int32),
            pltpu.VMEM((N // NW,), jnp.int32),
            pltpu.VMEM((N // NW // CH, CH), jnp.int32),
            pltpu.VMEM((16,), jnp.int32),
            pltpu.VMEM((16,), jnp.int32),
            pltpu.VMEM((CH, HIDDEN), jnp.float32),
            pltpu.VMEM((CH, HIDDEN), jnp.float32),
            pltpu.SemaphoreType.DMA,
            pltpu.SemaphoreType.DMA,
            pltpu.SemaphoreType.DMA,
            pltpu.SemaphoreType.DMA,
        ],
    )
    def k(ys_hbm, eid_hbm, rank_hbm, ps1_hbm, ps2_hbm, g_hbm,
          eid_v, rank_v, idx_buf, ps1_v, ps2_v,
          yb0, yb1, semg0, semg1, semw0, semw1):
        wid = lax.axis_index("s") * 2 + lax.axis_index("c")
        base = wid * BPW
        nch = BPW // CH
        ybuf = (yb0, yb1)
        semg = (semg0, semg1)
        semw = (semw0, semw1)
        pltpu.sync_copy(ps1_hbm, ps1_v)
        pltpu.sync_copy(ps2_hbm, ps2_v)
        pltpu.sync_copy(eid_hbm.at[pl.ds(base, BPW)], eid_v)
        pltpu.sync_copy(rank_hbm.at[pl.ds(base, BPW)], rank_v)
        _sc_idx_table(eid_v, rank_v, ps1_v, ps2_v, idx_buf, BPW)

        def ga(b, c):
            return pltpu.make_async_copy(ys_hbm.at[idx_buf.at[c]],
                                         ybuf[b], semg[b])

        def wr(b, c):
            row0 = base + c * CH
            return pltpu.make_async_copy(ybuf[b], g_hbm.at[pl.ds(row0, CH)],
                                         semw[b])

        ga(0, 0).start()
        ga(1, 1).start()

        @pl.loop(0, nch, step=2)
        def _(c0):
            for b in range(2):
                c = c0 + b
                ga(b, c).wait()
                wr(b, c).start()

                @pl.when(c + 2 < nch)
                def _():
                    wr(b, c).wait()
                    ga(b, c + 2).start()

        for b in range(2):
            wr(b, nch - 2 + b).wait()

    return k(ys, eid, rank, ps1, ps2)


def _expert_kernel(eids_ref, xs_ref, fcw_ref, fcb_ref, ow_ref, ob_ref,
                   ys_ref):
    xb = xs_ref[...].astype(jnp.bfloat16)
    h = jnp.dot(xb, fcw_ref[0], preferred_element_type=jnp.float32)
    h = h + fcb_ref[0]
    x1 = h[:, :FF]
    x2 = h[:, FF:]
    g = x1 * (0.5 * x2 * (1.0 + lax.erf(x2 * 0.7071067811865476)))
    y = jnp.dot(g.astype(jnp.bfloat16), ow_ref[0],
                preferred_element_type=jnp.float32)
    ys_ref[...] = y + ob_ref[0]


def _experts(xs, eids, fc_Wt, fc_b, out_Wt, out_b):
    P, d = xs.shape
    NT = P // T
    grid_spec = pltpu.PrefetchScalarGridSpec(
        num_scalar_prefetch=1,
        grid=(NT,),
        in_specs=[
            pl.BlockSpec((T, d), lambda i, eids: (i, 0)),
            pl.BlockSpec((1, d, 2 * FF), lambda i, eids: (eids[i], 0, 0)),
            pl.BlockSpec((1, 1, 2 * FF), lambda i, eids: (eids[i], 0, 0)),
            pl.BlockSpec((1, FF, d), lambda i, eids: (eids[i], 0, 0)),
            pl.BlockSpec((1, 1, d), lambda i, eids: (eids[i], 0, 0)),
        ],
        out_specs=pl.BlockSpec((T, d), lambda i, eids: (i, 0)),
    )
    return pl.pallas_call(
        _expert_kernel,
        grid_spec=grid_spec,
        out_shape=jax.ShapeDtypeStruct((P, d), jnp.float32),
    )(eids, xs, fc_Wt, fc_b.reshape(E, 1, 2 * FF), out_Wt,
      out_b.reshape(E, 1, HIDDEN))


def _scale_kernel(g_ref, w_ref, out_ref):
    wt = jnp.transpose(w_ref[0], (1, 0))
    out_ref[...] = g_ref[...] * wt


def _scale(gathered, w3):
    N, d = gathered.shape
    NB = N // TG
    return pl.pallas_call(
        _scale_kernel,
        grid=(NB,),
        in_specs=[
            pl.BlockSpec((TG, d), lambda i: (i, 0)),
            pl.BlockSpec((1, 1, TG), lambda i: (i, 0, 0)),
        ],
        out_specs=pl.BlockSpec((TG, d), lambda i: (i, 0)),
        out_shape=jax.ShapeDtypeStruct((N, d), jnp.float32),
    )(gathered, w3)


@jax.jit
def kernel(x, gate_W, gate_b, fc_W, fc_b, out_W, out_b):
    B, S, d = x.shape
    N = B * S
    P = N + E * T
    x_flat = x.reshape(N, d)
    gate_Wt = gate_W.T.astype(jnp.bfloat16)
    fc_Wt = fc_W.transpose(0, 2, 1).astype(jnp.bfloat16)
    out_Wt = out_W.transpose(0, 2, 1).astype(jnp.bfloat16)

    eid3, rank3, w3, counts, scores = _gating(x_flat, gate_Wt, gate_b)
    eid = eid3.reshape(N)
    rank = rank3.reshape(N)

    counts_i = counts.astype(jnp.int32)
    cpad = ((counts_i + T - 1) // T) * T
    ps1 = cpad[0]
    ps2 = cpad[0] + cpad[1]
    ps1v = jnp.full((16,), 1, jnp.int32) * ps1
    ps2v = jnp.full((16,), 1, jnp.int32) * ps2
    NT = P // T
    t_idx = jnp.arange(NT, dtype=jnp.int32)
    eids = ((t_idx >= ps1 // T).astype(jnp.int32)
            + (t_idx >= ps2 // T).astype(jnp.int32))

    xs = _dispatch(x_flat, eid, rank, ps1v, ps2v, P)
    ys = _experts(xs, eids, fc_Wt, fc_b, out_Wt, out_b)
    gathered = _combine(ys, eid, rank, ps1v, ps2v, N)
    out = _scale(gathered, w3)

    usage = scores / (counts + 1e-08)
    loss = jnp.sum((usage - 1.0 / E) ** 2)
    return out.reshape(B, S, d), loss
